# baseline (speedup 1.0000x reference)
import functools

import jax
import jax.numpy as jnp
from jax import lax
from jax.experimental import pallas as pl
from jax.experimental.pallas import tpu as pltpu
from jax.experimental.pallas import tpu_sc as plsc

VOCAB = 1_000_000
D = 64
P = 20
B = 1024
S = 200
OUT_S = P + S
NC = 2
NS = 16
NW = NC * NS
BPW = B // NW
CHUNKS = (64, 64, 64, 8)


def _gather_body(ids, table2, prompt, out,
                 idx_all, m_buf, tiles_a, tiles_b, obuf_a, obuf_b,
                 gsem_a, gsem_b, osem_a, osem_b):
    wid = lax.axis_index("s") * NC + lax.axis_index("c")
    b0 = wid * BPW
    lane = lax.iota(jnp.int32, 16)
    zeros16 = lane * 0

    pltpu.sync_copy(prompt, obuf_a.at[pl.ds(0, P)])
    pltpu.sync_copy(prompt, obuf_b.at[pl.ds(0, P)])

    bufs = ((obuf_a, osem_a), (obuf_b, osem_b))
    gbufs = ((tiles_a, gsem_a), (tiles_b, gsem_b))

    def select(tiles, obuf, off, csz):
        for g in range((csz + 15) // 16):
            v_vec = idx_all[pl.ds(off + 16 * g, 16)]
            for l in range(min(16, csz - 16 * g)):
                j = 16 * g + l
                half = (v_vec[l] & 1) * D
                for k in range(D // 16):
                    obuf[P + off + j, pl.ds(16 * k, 16)] = (
                        tiles[j, pl.ds(half + 16 * k, 16)])

    def start_chunk(gi, off, csz, moff):
        tiles, gsem = gbufs[gi]
        cpad = (csz + 15) // 16 * 16
        for g in range(cpad // 16):
            v = idx_all[pl.ds(off + 16 * g, 16)]
            m_buf[pl.ds(moff + 16 * g, 16)] = v >> 1
        return pltpu.async_copy(
            table2.at[m_buf.at[pl.ds(moff, cpad)]],
            tiles.at[pl.ds(0, cpad)], gsem)

    def pair(i2, carry):
        for u, (obuf, osem) in enumerate(bufs):
            b = b0 + 2 * i2 + u
            pltpu.sync_copy(ids.at[b], idx_all)
            idx_all[pl.ds(200, 16)] = zeros16

            @pl.when(2 * i2 + u >= 2)
            def _():
                pltpu.make_async_copy(
                    obuf.at[pl.ds(0, OUT_S)], out.at[b - 2], osem).wait()

            handles = [None, None]
            offs = []
            off = 0
            for ci, csz in enumerate(CHUNKS):
                offs.append(off)
                off += csz
            handle_prev = None
            for ci, csz in enumerate(CHUNKS):
                h = start_chunk(ci % 2, offs[ci], csz, 64 * (ci % 2))
                if ci > 0:
                    handle_prev.wait()
                    select(gbufs[(ci - 1) % 2][0], obuf,
                           offs[ci - 1], CHUNKS[ci - 1])
                handle_prev = h
            handle_prev.wait()
            select(gbufs[(len(CHUNKS) - 1) % 2][0], obuf,
                   offs[-1], CHUNKS[-1])
            pltpu.async_copy(obuf.at[pl.ds(0, OUT_S)], out.at[b], osem)
        return carry

    lax.fori_loop(0, BPW // 2, pair, 0)
    pltpu.make_async_copy(
        obuf_a.at[pl.ds(0, OUT_S)], out.at[b0 + BPW - 2], osem_a).wait()
    pltpu.make_async_copy(
        obuf_b.at[pl.ds(0, OUT_S)], out.at[b0 + BPW - 1], osem_b).wait()


_gather_call = functools.partial(
    pl.kernel,
    out_type=jax.ShapeDtypeStruct((B, OUT_S, D), jnp.float32),
    mesh=plsc.VectorSubcoreMesh(core_axis_name="c", subcore_axis_name="s"),
    scratch_types=[
        pltpu.VMEM((256,), jnp.int32),
        pltpu.VMEM((128,), jnp.int32),
        pltpu.VMEM((64, 2 * D), jnp.float32),
        pltpu.VMEM((64, 2 * D), jnp.float32),
        pltpu.VMEM((OUT_S, D), jnp.float32),
        pltpu.VMEM((OUT_S, D), jnp.float32),
        pltpu.SemaphoreType.DMA,
        pltpu.SemaphoreType.DMA,
        pltpu.SemaphoreType.DMA,
        pltpu.SemaphoreType.DMA,
    ],
)(_gather_body)


def _mask_body(am_ref, out_ref):
    out_ref[...] = jnp.concatenate(
        [jnp.ones((B, P), jnp.float32), am_ref[...]], axis=1)


def _mask_call(attention_mask):
    return pl.pallas_call(
        _mask_body,
        out_shape=jax.ShapeDtypeStruct((B, OUT_S), jnp.float32),
    )(attention_mask)


def kernel(input_ids, attention_mask, emb_table, prompt_table):
    ids = input_ids.astype(jnp.int32)
    ids = jnp.pad(ids, ((0, 0), (0, 256 - S)))
    table2 = emb_table.reshape(VOCAB // 2, 2 * D)
    emb_out = _gather_call(ids, table2, prompt_table)
    mask_out = _mask_call(attention_mask)
    return emb_out, mask_out
